# Initial kernel scaffold; baseline (speedup 1.0000x reference)
import jax
import jax.numpy as jnp
from jax.experimental import pallas as pl


def kernel(x, W1, g1, b1, W2, g2, b2, W3, g3, b3, W4, g4, b4, W5, g5, b5, Wl1, g6, b6, Wl2, bl2, g7, b7, Wl3, bl3):
    raise NotImplementedError("write your pallas kernel here")


# speedup vs baseline: 4.3730x; 4.3730x over previous
import functools

import jax
import jax.numpy as jnp
from jax import lax
from jax.experimental import pallas as pl
from jax.experimental.pallas import tpu as pltpu
from jax.experimental.pallas import tpu_sc as plsc

KNBR = 20
EPS = 1e-5


def _bf16_dot(a, b):
    return lax.dot_general(
        a.astype(jnp.bfloat16), b.astype(jnp.bfloat16),
        (((1,), (0,)), ((), ())), preferred_element_type=jnp.float32)


def _knn_body(xt_ref, gidx_ref, *, rb, n, k):
    b = pl.program_id(0)
    r = pl.program_id(1)
    xb = xt_ref[0]
    rows = xt_ref[0, pl.ds(r * rb, rb), :]
    g = lax.dot_general(
        rows.astype(jnp.bfloat16), xb.astype(jnp.bfloat16),
        (((1,), (1,)), ((), ())), preferred_element_type=jnp.float32)
    xx = jnp.sum(xb * xb, axis=1)
    xxr = jnp.sum(rows * rows, axis=1)
    pair = (2.0 * g - xxr[:, None]) - xx[None, :]
    ii = lax.broadcasted_iota(jnp.int32, (rb, n), 1)
    cols = []
    for _ in range(k):
        m = jnp.max(pair, axis=1, keepdims=True)
        cand = jnp.where(pair == m, ii, n)
        amin = jnp.min(cand, axis=1, keepdims=True)
        cols.append(amin)
        pair = jnp.where(ii == amin, -jnp.inf, pair)
    gidx_ref[0] = jnp.concatenate(cols, axis=1) + b * n


def _knn(xt, k):
    bsz, n, c = xt.shape
    rb = 256
    return pl.pallas_call(
        functools.partial(_knn_body, rb=rb, n=n, k=k),
        grid=(bsz, n // rb),
        in_specs=[pl.BlockSpec((1, n, c), lambda b, r: (b, 0, 0))],
        out_specs=pl.BlockSpec((1, rb, k), lambda b, r: (b, r, 0)),
        out_shape=jax.ShapeDtypeStruct((bsz, n, k), jnp.int32),
    )(xt)


def _sc_gather(table, idx_flat):
    rtot, c = table.shape
    e = idx_flat.shape[0]
    info = plsc.get_sparse_core_info()
    nc, ns = info.num_cores, info.num_subcores
    nw = nc * ns
    ch = 128
    per_w = e // nw
    n_chunks = per_w // ch
    mesh = plsc.VectorSubcoreMesh(core_axis_name="c", subcore_axis_name="s")

    @functools.partial(
        pl.kernel, mesh=mesh,
        compiler_params=pltpu.CompilerParams(use_tc_tiling_on_sc=False),
        out_type=jax.ShapeDtypeStruct((e, c), jnp.float32),
        scratch_types=[
            pltpu.VMEM((ch,), jnp.int32),
            pltpu.VMEM((ch, c), jnp.float32),
            pltpu.SemaphoreType.DMA,
        ],
    )
    def gather_k(table_hbm, idx_hbm, out_hbm, idx_v, rows_v, sem):
        wid = lax.axis_index("s") * nc + lax.axis_index("c")
        base0 = wid * per_w

        def body(i, carry):
            base = base0 + i * ch
            pltpu.sync_copy(idx_hbm.at[pl.ds(base, ch)], idx_v)
            pltpu.async_copy(table_hbm.at[idx_v], rows_v, sem).wait()
            pltpu.sync_copy(rows_v, out_hbm.at[pl.ds(base, ch)])
            return carry

        lax.fori_loop(0, n_chunks, body, 0)

    return gather_k(table, idx_flat)


def _edge_body(fe_ref, x_ref, w_ref, mpre_ref, sums_ref, comp_ref, *, p, k):
    i = pl.program_id(0)
    c = x_ref.shape[1]
    o = w_ref.shape[1]
    fe = fe_ref[...]
    xb = x_ref[...]
    d = fe.reshape(p, k, c) - xb[:, None, :]
    xk = jnp.broadcast_to(xb[:, None, :], (p, k, c))
    hcat = jnp.concatenate([d, xk], axis=2).reshape(p * k, 2 * c)
    h = _bf16_dot(hcat, w_ref[...])
    h3 = h.reshape(p, k, o)
    mpre_ref[...] = jnp.max(h3, axis=1)
    s1 = jnp.sum(jnp.sum(h3, axis=1), axis=0)
    s2 = jnp.sum(jnp.sum(h3 * h3, axis=1), axis=0)
    part = jnp.stack([s1, s2], axis=0)

    @pl.when(i == 0)
    def _():
        sums_ref[...] = jnp.zeros_like(sums_ref)
        comp_ref[...] = jnp.zeros_like(comp_ref)

    y = part - comp_ref[...]
    t = sums_ref[...] + y
    comp_ref[...] = (t - sums_ref[...]) - y
    sums_ref[...] = t


def _edgeconv(fe, xt, wcat, k):
    rtot, c = xt.shape
    o = wcat.shape[1]
    p = 128
    grid = (rtot // p,)
    return pl.pallas_call(
        functools.partial(_edge_body, p=p, k=k),
        grid=grid,
        in_specs=[
            pl.BlockSpec((p * k, c), lambda i: (i, 0)),
            pl.BlockSpec((p, c), lambda i: (i, 0)),
            pl.BlockSpec(wcat.shape, lambda i: (0, 0)),
        ],
        out_specs=[
            pl.BlockSpec((p, o), lambda i: (i, 0)),
            pl.BlockSpec((2, o), lambda i: (0, 0)),
        ],
        out_shape=[
            jax.ShapeDtypeStruct((rtot, o), jnp.float32),
            jax.ShapeDtypeStruct((2, o), jnp.float32),
        ],
        scratch_shapes=[pltpu.VMEM((2, o), jnp.float32)],
    )(fe, xt, wcat)


def _apply_body(mpre_ref, sums_ref, gm_ref, bt_ref, out_ref, *, cnt):
    s1 = sums_ref[0:1, :]
    s2 = sums_ref[1:2, :]
    mean = s1 / cnt
    var = s2 / cnt - mean * mean
    t = (mpre_ref[...] - mean) / jnp.sqrt(var + EPS) * gm_ref[...] \
        + bt_ref[...]
    out_ref[...] = jnp.where(t >= 0, t, 0.2 * t)


def _bn_apply(mpre, sums, gamma, beta, cnt):
    rtot, o = mpre.shape
    p = 1024
    return pl.pallas_call(
        functools.partial(_apply_body, cnt=cnt),
        grid=(rtot // p,),
        in_specs=[
            pl.BlockSpec((p, o), lambda i: (i, 0)),
            pl.BlockSpec((2, o), lambda i: (0, 0)),
            pl.BlockSpec((1, o), lambda i: (0, 0)),
            pl.BlockSpec((1, o), lambda i: (0, 0)),
        ],
        out_specs=pl.BlockSpec((p, o), lambda i: (i, 0)),
        out_shape=jax.ShapeDtypeStruct((rtot, o), jnp.float32),
    )(mpre, sums, gamma.reshape(1, o), beta.reshape(1, o))


def _h1_body(x1_ref, x2_ref, x3_ref, x4_ref, w_ref, g_ref, sums_ref,
             comp_ref):
    i = pl.program_id(0)
    rows = jnp.concatenate(
        [x1_ref[...], x2_ref[...], x3_ref[...], x4_ref[...]], axis=1)
    gb = _bf16_dot(rows, w_ref[...])
    g_ref[...] = gb
    s1 = jnp.sum(gb, axis=0)
    s2 = jnp.sum(gb * gb, axis=0)
    part = jnp.stack([s1, s2], axis=0)

    @pl.when(i == 0)
    def _():
        sums_ref[...] = jnp.zeros_like(sums_ref)
        comp_ref[...] = jnp.zeros_like(comp_ref)

    y = part - comp_ref[...]
    t = sums_ref[...] + y
    comp_ref[...] = (t - sums_ref[...]) - y
    sums_ref[...] = t


def _head1(x1, x2, x3, x4, w5t):
    rtot = x1.shape[0]
    emb = w5t.shape[1]
    p = 512
    return pl.pallas_call(
        _h1_body,
        grid=(rtot // p,),
        in_specs=[
            pl.BlockSpec((p, x1.shape[1]), lambda i: (i, 0)),
            pl.BlockSpec((p, x2.shape[1]), lambda i: (i, 0)),
            pl.BlockSpec((p, x3.shape[1]), lambda i: (i, 0)),
            pl.BlockSpec((p, x4.shape[1]), lambda i: (i, 0)),
            pl.BlockSpec(w5t.shape, lambda i: (0, 0)),
        ],
        out_specs=[
            pl.BlockSpec((p, emb), lambda i: (i, 0)),
            pl.BlockSpec((2, emb), lambda i: (0, 0)),
        ],
        out_shape=[
            jax.ShapeDtypeStruct((rtot, emb), jnp.float32),
            jax.ShapeDtypeStruct((2, emb), jnp.float32),
        ],
        scratch_shapes=[pltpu.VMEM((2, emb), jnp.float32)],
    )(x1, x2, x3, x4, w5t)


def _h2_body(g_ref, sums_ref, gm_ref, bt_ref, p_ref, *, n, cnt):
    s1 = sums_ref[0:1, :]
    s2 = sums_ref[1:2, :]
    mean = s1 / cnt
    var = s2 / cnt - mean * mean
    t = (g_ref[0] - mean) / jnp.sqrt(var + EPS) * gm_ref[...] + bt_ref[...]
    t = jnp.where(t >= 0, t, 0.2 * t)
    pmax = jnp.max(t, axis=0, keepdims=True)
    pmean = jnp.sum(t, axis=0, keepdims=True) / n
    p_ref[0] = jnp.concatenate([pmax, pmean], axis=1)


def _head2(g, sums, gamma, beta, bsz, n, cnt):
    emb = g.shape[2]
    return pl.pallas_call(
        functools.partial(_h2_body, n=n, cnt=cnt),
        grid=(bsz,),
        in_specs=[
            pl.BlockSpec((1, n, emb), lambda b: (b, 0, 0)),
            pl.BlockSpec((2, emb), lambda b: (0, 0)),
            pl.BlockSpec((1, emb), lambda b: (0, 0)),
            pl.BlockSpec((1, emb), lambda b: (0, 0)),
        ],
        out_specs=pl.BlockSpec((1, 1, 2 * emb), lambda b: (b, 0, 0)),
        out_shape=jax.ShapeDtypeStruct((bsz, 1, 2 * emb), jnp.float32),
    )(g, sums, gamma.reshape(1, emb), beta.reshape(1, emb))


def _bn_rows(h, gamma, beta):
    mu = jnp.mean(h, axis=0, keepdims=True)
    var = jnp.mean((h - mu) * (h - mu), axis=0, keepdims=True)
    return (h - mu) / jnp.sqrt(var + EPS) * gamma + beta


def _h3_body(p_ref, wl1_ref, g6_ref, b6_ref, wl2_ref, bl2_ref, g7_ref,
             b7_ref, wl3_ref, bl3_ref, out_ref):
    h = _bf16_dot(p_ref[...], wl1_ref[...])
    h = _bn_rows(h, g6_ref[...], b6_ref[...])
    h = jnp.where(h >= 0, h, 0.2 * h)
    h = _bf16_dot(h, wl2_ref[...]) + bl2_ref[...]
    h = _bn_rows(h, g7_ref[...], b7_ref[...])
    h = jnp.where(h >= 0, h, 0.2 * h)
    out_ref[...] = _bf16_dot(h, wl3_ref[...]) + bl3_ref[...]


def _head3(p, wl1, g6, b6, wl2, bl2, g7, b7, wl3, bl3):
    bsz = p.shape[0]
    nc = wl3.shape[1]
    args = [p, wl1, g6.reshape(1, -1), b6.reshape(1, -1), wl2,
            bl2.reshape(1, -1), g7.reshape(1, -1), b7.reshape(1, -1),
            wl3, bl3.reshape(1, -1)]
    return pl.pallas_call(
        _h3_body,
        in_specs=[pl.BlockSpec(a.shape, lambda: tuple(0 for _ in a.shape))
                  for a in args],
        out_specs=pl.BlockSpec((bsz, nc), lambda: (0, 0)),
        out_shape=jax.ShapeDtypeStruct((bsz, nc), jnp.float32),
    )(*args)


def _prep_w(w, c, cpad):
    wn = w[:, :c].T
    wc = w[:, c:].T
    if cpad > c:
        wn = jnp.pad(wn, ((0, cpad - c), (0, 0)))
        wc = jnp.pad(wc, ((0, cpad - c), (0, 0)))
    return jnp.concatenate([wn, wc], axis=0)


def kernel(x, W1, g1, b1, W2, g2, b2, W3, g3, b3, W4, g4, b4, W5, g5, b5,
           Wl1, g6, b6, Wl2, bl2, g7, b7, Wl3, bl3):
    bsz, n, _ = x.shape
    rtot = bsz * n
    cnt = float(rtot * KNBR)

    xt = jnp.pad(x, ((0, 0), (0, 0), (0, 13)))
    feats = []
    cur = xt
    cin_real, cin = 3, 16
    for (w, gm, bt) in [(W1, g1, b1), (W2, g2, b2), (W3, g3, b3),
                        (W4, g4, b4)]:
        o = w.shape[0]
        wcat = _prep_w(w, cin_real, cin)
        gidx = _knn(cur, KNBR)
        table = cur.reshape(rtot, cin)
        fe = _sc_gather(table, gidx.reshape(rtot * KNBR))
        mpre, sums = _edgeconv(fe, table, wcat, KNBR)
        xn = _bn_apply(mpre, sums, gm, bt, cnt)
        feats.append(xn)
        cur = xn.reshape(bsz, n, o)
        cin_real = cin = o

    g, sums5 = _head1(feats[0], feats[1], feats[2], feats[3], W5.T)
    emb = W5.shape[0]
    p = _head2(g.reshape(bsz, n, emb), sums5, g5, b5, bsz, n, float(rtot))
    return _head3(p.reshape(bsz, 2 * emb), Wl1, g6, b6, Wl2, bl2, g7, b7,
                  Wl3, bl3)
